# baseline (speedup 1.0000x reference)
import functools

import jax
import jax.numpy as jnp
from jax import lax
from jax.experimental import pallas as pl
from jax.experimental.pallas import tpu as pltpu, tpu_sc as plsc

D_MODEL_ = 4096
N_EXPERTS_ = 64
TOP_K_ = 8
BT_ = 1024
NW_ = 32
GRP_ = BT_ // 16


def _gate_block(x_ref, w_ref, probs_ref, psum_ref, acc_ref):
    i = pl.program_id(0)
    nsteps = pl.num_programs(0)

    @pl.when(i == 0)
    def _init():
        acc_ref[...] = jnp.zeros_like(acc_ref)

    logits = jax.lax.dot_general(
        w_ref[...], x_ref[...],
        dimension_numbers=(((1,), (1,)), ((), ())),
        preferred_element_type=jnp.float32,
    )
    m = jnp.max(logits, axis=0, keepdims=True)
    e = jnp.exp(logits - m)
    s = jnp.sum(e, axis=0, keepdims=True)
    probs = e / s
    probs_ref[...] = probs[None]
    acc_ref[...] += jnp.sum(probs, axis=1, keepdims=True)

    @pl.when(i == nsteps - 1)
    def _finish():
        psum_ref[...] = acc_ref[...]


def _sc_topk(probs_hbm, vals_hbm, idx_hbm, cnts_hbm, buf, vbuf, ibuf, cnt):
    wid = lax.axis_index("s") * 2 + lax.axis_index("c")
    pltpu.sync_copy(probs_hbm.at[wid], buf)

    for i in range(N_EXPERTS_ // 16):
        cnt[pl.ds(i * 16, 16)] = jnp.zeros((16,), jnp.float32)

    iota = lax.iota(jnp.int32, 16)
    ones = jnp.ones((16,), jnp.float32)

    def body(g, carry):
        base = g * 16
        work = []
        for e in range(N_EXPERTS_):
            v = buf[e, pl.ds(base, 16)]
            b = lax.bitcast_convert_type(v, jnp.int32)
            work.append((b & -64) | (63 - e))
        rows = (base + iota) * TOP_K_
        for j in range(TOP_K_):
            mx = work[0]
            for e in range(1, N_EXPERTS_):
                mx = jnp.maximum(mx, work[e])
            idxv = 63 - (mx & 63)
            valv = lax.bitcast_convert_type(mx & -64, jnp.float32)
            flat = rows + j
            plsc.store_scatter(vbuf, [flat], valv)
            plsc.store_scatter(ibuf, [flat], idxv)
            plsc.addupdate_scatter(cnt, [idxv], ones)
            if j != TOP_K_ - 1:
                work = [jnp.where(w == mx, -1, w) for w in work]
        return carry

    lax.fori_loop(0, GRP_, body, 0)

    pltpu.sync_copy(vbuf, vals_hbm.at[pl.ds(wid * (BT_ * TOP_K_), BT_ * TOP_K_)])
    pltpu.sync_copy(ibuf, idx_hbm.at[pl.ds(wid * (BT_ * TOP_K_), BT_ * TOP_K_)])
    pltpu.sync_copy(cnt, cnts_hbm.at[wid])


@functools.partial(jax.jit, static_argnames=())
def kernel(x, W):
    B, T, D = x.shape
    n_tok = B * T
    x2 = x.reshape(n_tok, D)
    grid = (n_tok // BT_,)

    probsP, psum = pl.pallas_call(
        _gate_block,
        grid=grid,
        in_specs=[
            pl.BlockSpec((BT_, D), lambda i: (i, 0)),
            pl.BlockSpec((N_EXPERTS_, D), lambda i: (0, 0)),
        ],
        out_specs=[
            pl.BlockSpec((1, N_EXPERTS_, BT_), lambda i: (i, 0, 0)),
            pl.BlockSpec((N_EXPERTS_, 1), lambda i: (0, 0)),
        ],
        out_shape=[
            jax.ShapeDtypeStruct((n_tok // BT_, N_EXPERTS_, BT_), jnp.float32),
            jax.ShapeDtypeStruct((N_EXPERTS_, 1), jnp.float32),
        ],
        scratch_shapes=[pltpu.VMEM((N_EXPERTS_, 1), jnp.float32)],
    )(x2, W)

    mesh = plsc.VectorSubcoreMesh(core_axis_name="c", subcore_axis_name="s")
    vals, idx, cnts = pl.kernel(
        _sc_topk,
        out_type=[
            jax.ShapeDtypeStruct((n_tok * TOP_K_,), jnp.float32),
            jax.ShapeDtypeStruct((n_tok * TOP_K_,), jnp.int32),
            jax.ShapeDtypeStruct((NW_, N_EXPERTS_), jnp.float32),
        ],
        mesh=mesh,
        compiler_params=pltpu.CompilerParams(needs_layout_passes=False),
        scratch_types=[
            pltpu.VMEM((N_EXPERTS_, BT_), jnp.float32),
            pltpu.VMEM((BT_ * TOP_K_,), jnp.float32),
            pltpu.VMEM((BT_ * TOP_K_,), jnp.int32),
            pltpu.VMEM((N_EXPERTS_,), jnp.float32),
        ],
    )(probsP)

    scale = 1.0 / (float(n_tok) * float(TOP_K_) * float(n_tok))
    loss = jnp.sum(cnts.sum(axis=0) * psum[:, 0]) * scale
    return (vals.reshape(B, T, TOP_K_), idx.reshape(B, T, TOP_K_), loss)
